# Initial kernel scaffold; baseline (speedup 1.0000x reference)
import jax
import jax.numpy as jnp
from jax.experimental import pallas as pl


def kernel(tokens, tables):
    raise NotImplementedError("write your pallas kernel here")


# speedup vs baseline: 4.7724x; 4.7724x over previous
import functools

import jax
import jax.numpy as jnp
from jax import lax
from jax.experimental import pallas as pl
from jax.experimental.pallas import tpu as pltpu
from jax.experimental.pallas import tpu_sc as plsc

NUM_Q = 8
VOCAB = 1024
DIM = 256
B = 16
T = 2048

NC, NS = 2, 16
NW = NC * NS
ROWS = B * T
RPW = ROWS // NW
CHUNK = 128
NCHUNK = RPW // CHUNK
NSLOT = 3

_OUT_TYPE = tuple(
    jax.ShapeDtypeStruct((ROWS, DIM), jnp.float32) for _ in range(NUM_Q)
)


def _body(tab_hbm, idx_hbm, *refs):
    outs = refs[:NUM_Q]
    idx_v, buf, gsems, wsems = refs[NUM_Q:]

    w = lax.axis_index("s") * NC + lax.axis_index("c")
    b = w // 2
    h = w % 2

    for q in range(NUM_Q):
        src_row = b * (NUM_Q * T // CHUNK) + q * (T // CHUNK) + h * NCHUNK
        pltpu.sync_copy(
            idx_hbm.at[pl.ds(src_row, NCHUNK)],
            idx_v.at[pl.ds(q * NCHUNK, NCHUNK)],
        )

    row0 = w * RPW

    steps = [(q, cc) for q in range(NUM_Q) for cc in range(NCHUNK)]
    n = len(steps)
    ghandles = [None] * NSLOT
    whandles = [None] * NSLOT

    def _writeback(kp):
        qp, ccp = steps[kp]
        sp = kp % NSLOT
        ghandles[sp].wait()
        whandles[sp] = pltpu.async_copy(
            buf.at[sp],
            outs[qp].at[pl.ds(row0 + ccp * CHUNK, CHUNK)],
            wsems.at[sp],
        )

    for k, (q, cc) in enumerate(steps):
        s = k % NSLOT
        if whandles[s] is not None:
            whandles[s].wait()
        ghandles[s] = pltpu.async_copy(
            tab_hbm.at[idx_v.at[q * NCHUNK + cc]],
            buf.at[s],
            gsems.at[s],
        )
        if k >= NSLOT - 1:
            _writeback(k - (NSLOT - 1))
    for kp in range(max(0, n - (NSLOT - 1)), n):
        _writeback(kp)
    for sp in range(NSLOT):
        if whandles[sp] is not None:
            whandles[sp].wait()


_sc_gather = functools.partial(
    pl.kernel,
    out_type=_OUT_TYPE,
    mesh=plsc.VectorSubcoreMesh(core_axis_name="c", subcore_axis_name="s"),
    scratch_types=[
        pltpu.VMEM((NUM_Q * NCHUNK, CHUNK), jnp.int32),
        pltpu.VMEM((NSLOT, CHUNK, DIM), jnp.float32),
        pltpu.SemaphoreType.DMA((NSLOT,)),
        pltpu.SemaphoreType.DMA((NSLOT,)),
    ],
)(_body)


def kernel(tokens, tables):
    offs = (jnp.arange(NUM_Q, dtype=jnp.int32) * VOCAB)[None, :, None]
    idx = (tokens.astype(jnp.int32) + offs).reshape(B * NUM_Q * T // CHUNK, CHUNK)
    tab = tables.reshape(NUM_Q * VOCAB, DIM)
    outs = _sc_gather(tab, idx)
    return tuple(o.reshape(B, T, DIM) for o in outs)
